# Initial kernel scaffold; baseline (speedup 1.0000x reference)
import jax
import jax.numpy as jnp
from jax.experimental import pallas as pl


def kernel(ability, labels, W_wkr, b_wkr, W_tsk, b_tsk, edge_weight, src_tw, dst_tw, etype_tw, src_wt, dst_wt, etype_wt):
    raise NotImplementedError("write your pallas kernel here")


# speedup vs baseline: 1.9094x; 1.9094x over previous
import functools

import jax
import jax.numpy as jnp
from jax import lax
from jax.experimental import pallas as pl
from jax.experimental.pallas import tpu as pltpu
from jax.experimental.pallas import tpu_sc as plsc

NUM_CORES = 2
NUM_SUBCORES = 16
NSUB = NUM_CORES * NUM_SUBCORES
K = 128
L = 16


def _sc_gather_segsum(table, src, dst, etype, ew, n_out):
    E = src.shape[0]
    D = table.shape[1]
    npw = ((n_out + NSUB - 1) // NSUB + 7) // 8 * 8
    n_pad = NSUB * npw

    e_pad = ((E + K - 1) // K) * K + K
    pad = e_pad - E
    src_p = jnp.pad(src, (0, pad))
    dst_p = jnp.pad(dst, (0, pad))
    et_p = jnp.pad(etype, (0, pad))

    bounds = (jnp.arange(NSUB + 1, dtype=jnp.int32) * npw).astype(jnp.int32)
    es = jnp.searchsorted(dst, bounds, side="left").astype(jnp.int32)
    es = jnp.pad(es, (0, 64 - (NSUB + 1)))

    R = ew.shape[0]
    r_pad = ((R + L - 1) // L) * L + L
    ew_p = jnp.pad(ew, (0, r_pad - R))

    mesh = plsc.VectorSubcoreMesh(
        core_axis_name="c", subcore_axis_name="s",
        num_cores=NUM_CORES, num_subcores=NUM_SUBCORES)

    @functools.partial(
        pl.kernel,
        mesh=mesh,
        out_type=(jax.ShapeDtypeStruct((n_pad, D), jnp.float32),
                  jax.ShapeDtypeStruct((n_pad, L), jnp.float32)),
        scratch_types=[
            pltpu.VMEM((K,), jnp.int32),
            pltpu.VMEM((K, D), jnp.float32),
            pltpu.VMEM((npw, D), jnp.float32),
            pltpu.VMEM((npw, L), jnp.float32),
            pltpu.VMEM((K + L,), jnp.int32),
            pltpu.VMEM((K + L,), jnp.int32),
            pltpu.VMEM((64,), jnp.int32),
            pltpu.VMEM((r_pad,), jnp.float32),
            pltpu.SemaphoreType.DMA,
        ],
    )
    def k(table_h, src_h, dst_h, et_h, ew_h, es_h, agg_h, deg_h,
          idx_v, rows_v, acc_v, deg_v, dst_s, et_s, es_s, ew_s, sem):
        wid = lax.axis_index("s") * NUM_CORES + lax.axis_index("c")
        nbase = wid * npw
        pltpu.sync_copy(es_h, es_s)
        pltpu.sync_copy(ew_h, ew_s)
        es_pair = es_s[pl.ds(wid, L)]
        estart = es_pair[0]
        eend = es_pair[1]

        zero = jnp.zeros((L,), jnp.float32)

        def zbody(i, carry):
            for j in range(D // L):
                acc_v[i, pl.ds(j * L, L)] = zero
            deg_v[i, pl.ds(0, L)] = zero
            return carry
        lax.fori_loop(0, npw, zbody, 0)

        e0a = (estart // 8) * 8
        nchunks = (eend - e0a + (K - 1)) // K

        def chunk_body(c, carry):
            cb = e0a + c * K
            pltpu.sync_copy(src_h.at[pl.ds(cb, K)], idx_v)
            pltpu.sync_copy(dst_h.at[pl.ds(cb, K)], dst_s.at[pl.ds(0, K)])
            pltpu.sync_copy(et_h.at[pl.ds(cb, K)], et_s.at[pl.ds(0, K)])
            pltpu.async_copy(table_h.at[idx_v], rows_v, sem).wait()
            lo = jnp.maximum(estart - cb, 0)
            hi = jnp.minimum(eend - cb, K)

            def edge_body(e, ecarry):
                d = dst_s[pl.ds(e, L)][0] - nbase
                et = et_s[pl.ds(e, L)][0]
                w = ew_s[pl.ds(et, L)][0]
                wv = jnp.full((L,), w, jnp.float32)
                deg_v[d, pl.ds(0, L)] = deg_v[d, pl.ds(0, L)] + 1.0
                for j in range(D // L):
                    sl = pl.ds(j * L, L)
                    acc_v[d, sl] = acc_v[d, sl] + rows_v[e, sl] * wv
                return ecarry
            lax.fori_loop(lo, hi, edge_body, 0)
            return carry
        lax.fori_loop(0, nchunks, chunk_body, 0)

        pltpu.sync_copy(acc_v, agg_h.at[pl.ds(nbase, npw)])
        pltpu.sync_copy(deg_v, deg_h.at[pl.ds(nbase, npw)])

    return k(table, src_p, dst_p, et_p, ew_p, es)


def _tc_update(x, agg, deg, W, b):
    N, D = x.shape
    W1 = W[:D]
    W2 = W[D:]
    b2 = b.reshape(1, -1)
    Dout = W.shape[1]
    RB = 1000
    assert N % RB == 0

    def body(x_ref, a_ref, dg_ref, w1_ref, w2_ref, b_ref, o_ref):
        deg_col = jnp.maximum(dg_ref[:, 0:1], 1.0)
        msg = a_ref[:] / deg_col
        acc = jnp.dot(x_ref[:], w1_ref[:], preferred_element_type=jnp.float32)
        acc = acc + jnp.dot(msg, w2_ref[:], preferred_element_type=jnp.float32)
        o_ref[:] = jnp.maximum(acc + b_ref[:], 0.0)

    return pl.pallas_call(
        body,
        grid=(N // RB,),
        in_specs=[
            pl.BlockSpec((RB, D), lambda i: (i, 0)),
            pl.BlockSpec((RB, D), lambda i: (i, 0)),
            pl.BlockSpec((RB, L), lambda i: (i, 0)),
            pl.BlockSpec((D, Dout), lambda i: (0, 0)),
            pl.BlockSpec((D, Dout), lambda i: (0, 0)),
            pl.BlockSpec((1, Dout), lambda i: (0, 0)),
        ],
        out_specs=pl.BlockSpec((RB, Dout), lambda i: (i, 0)),
        out_shape=jax.ShapeDtypeStruct((N, Dout), jnp.float32),
    )(x, agg[:N], deg[:N], W1, W2, b2)


def kernel(ability, labels, W_wkr, b_wkr, W_tsk, b_tsk, edge_weight,
           src_tw, dst_tw, etype_tw, src_wt, dst_wt, etype_wt):
    n_wkr = ability.shape[0]
    n_tsk = labels.shape[0]

    agg_w, deg_w = _sc_gather_segsum(labels, src_tw, dst_tw, etype_tw,
                                     edge_weight, n_wkr)
    ability_new = _tc_update(ability, agg_w, deg_w, W_wkr, b_wkr)

    agg_t, deg_t = _sc_gather_segsum(ability_new, src_wt, dst_wt, etype_wt,
                                     edge_weight, n_tsk)
    labels_new = _tc_update(labels, agg_t, deg_t, W_tsk, b_tsk)

    return ability_new, labels_new


# speedup vs baseline: 2.9906x; 1.5663x over previous
import functools

import jax
import jax.numpy as jnp
from jax import lax
from jax.experimental import pallas as pl
from jax.experimental.pallas import tpu as pltpu
from jax.experimental.pallas import tpu_sc as plsc

NUM_CORES = 2
NUM_SUBCORES = 16
NSUB = NUM_CORES * NUM_SUBCORES
K = 128
L = 16


C = 256
G = C // L


def _sc_gather_segsum(table, src, dst, etype, ew, n_out):
    E = src.shape[0]
    D = table.shape[1]
    npw = ((n_out + NSUB - 1) // NSUB + 7) // 8 * 8
    n_pad = NSUB * npw

    e_pad = ((E + C - 1) // C) * C + 2 * C
    pad = e_pad - E
    src_p = jnp.pad(src, (0, pad))
    dst_p = jnp.pad(dst, (0, pad))
    et_p = jnp.pad(etype, (0, pad))

    bounds = (jnp.arange(NSUB + 1, dtype=jnp.int32) * npw).astype(jnp.int32)
    es = jnp.searchsorted(dst, bounds, side="left").astype(jnp.int32)
    es = jnp.pad(es, (0, 64 - (NSUB + 1)))

    R = ew.shape[0]
    r_pad = ((R + L - 1) // L) * L + L
    ew_p = jnp.pad(ew, (0, r_pad - R))

    mesh = plsc.VectorSubcoreMesh(
        core_axis_name="c", subcore_axis_name="s",
        num_cores=NUM_CORES, num_subcores=NUM_SUBCORES)

    @functools.partial(
        pl.kernel,
        mesh=mesh,
        out_type=(jax.ShapeDtypeStruct((n_pad, D), jnp.float32),
                  jax.ShapeDtypeStruct((n_pad, L), jnp.float32)),
        scratch_types=[
            pltpu.VMEM((C,), jnp.int32),
            pltpu.VMEM((C,), jnp.int32),
            pltpu.VMEM((C,), jnp.int32),
            pltpu.VMEM((C,), jnp.int32),
            pltpu.VMEM((C,), jnp.int32),
            pltpu.VMEM((C,), jnp.int32),
            pltpu.VMEM((C, D), jnp.float32),
            pltpu.VMEM((C, D), jnp.float32),
            pltpu.VMEM((npw, D), jnp.float32),
            pltpu.VMEM((npw, L), jnp.float32),
            pltpu.VMEM((64,), jnp.int32),
            pltpu.VMEM((r_pad,), jnp.float32),
            pltpu.SemaphoreType.DMA,
            pltpu.SemaphoreType.DMA,
            pltpu.SemaphoreType.DMA,
            pltpu.SemaphoreType.DMA,
            pltpu.SemaphoreType.DMA,
            pltpu.SemaphoreType.DMA,
        ],
    )
    def k(table_h, src_h, dst_h, et_h, ew_h, es_h, agg_h, deg_h,
          src0, src1, dst0, dst1, et0, et1, rows0, rows1,
          acc_v, deg_v, es_s, ew_s,
          sem_s0, sem_s1, sem_m0, sem_m1, sem_r0, sem_r1):
        srcb = (src0, src1)
        dstb = (dst0, dst1)
        etb = (et0, et1)
        rowsb = (rows0, rows1)
        sem_s = (sem_s0, sem_s1)
        sem_m = (sem_m0, sem_m1)
        sem_r = (sem_r0, sem_r1)

        wid = lax.axis_index("s") * NUM_CORES + lax.axis_index("c")
        nbase = wid * npw
        pltpu.sync_copy(es_h, es_s)
        pltpu.sync_copy(ew_h, ew_s)
        es_pair = es_s[pl.ds(wid, L)]
        estart = es_pair[0]
        eend = es_pair[1]

        zero = jnp.zeros((L,), jnp.float32)

        def zbody(i, carry):
            for j in range(D // L):
                acc_v[i, pl.ds(j * L, L)] = zero
            deg_v[i, pl.ds(0, L)] = zero
            return carry
        lax.fori_loop(0, npw, zbody, 0)

        e0a = (estart // 8) * 8
        nchunks = (eend - e0a + (C - 1)) // C

        def fire_src(chunk, b):
            cb = e0a + chunk * C
            pltpu.async_copy(src_h.at[pl.ds(cb, C)], srcb[b], sem_s[b])

        def drain_src(chunk, b):
            cb = e0a + chunk * C
            pltpu.make_async_copy(
                src_h.at[pl.ds(cb, C)], srcb[b], sem_s[b]).wait()

        def fire_meta(chunk, b):
            cb = e0a + chunk * C
            pltpu.async_copy(dst_h.at[pl.ds(cb, C)], dstb[b], sem_m[b])
            pltpu.async_copy(et_h.at[pl.ds(cb, C)], etb[b], sem_m[b])

        def drain_meta(chunk, b):
            cb = e0a + chunk * C
            pltpu.make_async_copy(
                dst_h.at[pl.ds(cb, C)], dstb[b], sem_m[b]).wait()
            pltpu.make_async_copy(
                et_h.at[pl.ds(cb, C)], etb[b], sem_m[b]).wait()

        def fire_rows(b):
            for j in range(C // K):
                pltpu.async_copy(
                    table_h.at[srcb[b].at[pl.ds(j * K, K)]],
                    rowsb[b].at[pl.ds(j * K, K)], sem_r[b])

        def drain_rows(b):
            for j in range(C // K):
                pltpu.make_async_copy(
                    table_h.at[srcb[b].at[pl.ds(j * K, K)]],
                    rowsb[b].at[pl.ds(j * K, K)], sem_r[b]).wait()

        def compute(c, b):
            cb = e0a + c * C
            lo_rel = estart - cb
            hi_rel = eend - cb
            rows_v = rowsb[b]
            dst_s = dstb[b]
            et_s = etb[b]
            ew_vec = ew_s[pl.ds(0, L)]

            def group_body(g, carry):
                base = g * L
                dst16 = dst_s[pl.ds(base, L)]
                et16 = et_s[pl.ds(base, L)]
                w16 = lax.gather(
                    ew_vec, et16[:, None],
                    lax.GatherDimensionNumbers(
                        offset_dims=(), collapsed_slice_dims=(0,),
                        start_index_map=(0,)),
                    slice_sizes=(1,),
                    mode=lax.GatherScatterMode.PROMISE_IN_BOUNDS)
                rel = lax.iota(jnp.int32, L) + base
                mask = (rel >= lo_rel) & (rel < hi_rel)
                wm = jnp.where(mask, w16, 0.0)
                ones = jnp.where(mask, 1.0, 0.0)
                dm = jnp.clip(dst16 - nbase, 0, npw - 1)
                for kk in range(L):
                    d = dm[kk]
                    e = base + kk
                    wv = jnp.full((L,), wm[kk], jnp.float32)
                    deg_v[d, pl.ds(0, L)] = (
                        deg_v[d, pl.ds(0, L)]
                        + jnp.full((L,), ones[kk], jnp.float32))
                    for j in range(D // L):
                        sl = pl.ds(j * L, L)
                        acc_v[d, sl] = acc_v[d, sl] + rows_v[e, sl] * wv
                return carry
            lax.fori_loop(0, G, group_body, 0)

        @pl.when(nchunks >= 1)
        def _():
            fire_src(0, 0)

        @pl.when(nchunks >= 2)
        def _():
            fire_src(1, 1)

        @pl.when(nchunks >= 1)
        def _():
            drain_src(0, 0)
            fire_meta(0, 0)
            fire_rows(0)

        def pair_body(c2, carry):
            for b in range(2):
                c = c2 * 2 + b
                nb = 1 - b

                @pl.when(c < nchunks)
                def _():
                    @pl.when(c + 1 < nchunks)
                    def _():
                        fire_meta(c + 1, nb)
                        drain_src(c + 1, nb)
                        fire_rows(nb)

                    drain_rows(b)
                    drain_meta(c, b)

                    @pl.when(c + 2 < nchunks)
                    def _():
                        fire_src(c + 2, b)

                    compute(c, b)
            return carry
        lax.fori_loop(0, (nchunks + 1) // 2, pair_body, 0)

        pltpu.sync_copy(acc_v, agg_h.at[pl.ds(nbase, npw)])
        pltpu.sync_copy(deg_v, deg_h.at[pl.ds(nbase, npw)])

    return k(table, src_p, dst_p, et_p, ew_p, es)


def _tc_update(x, agg, deg, W, b):
    N, D = x.shape
    W1 = W[:D]
    W2 = W[D:]
    b2 = b.reshape(1, -1)
    Dout = W.shape[1]
    RB = 1000
    assert N % RB == 0

    def body(x_ref, a_ref, dg_ref, w1_ref, w2_ref, b_ref, o_ref):
        deg_col = jnp.maximum(dg_ref[:, 0:1], 1.0)
        msg = a_ref[:] / deg_col
        acc = jnp.dot(x_ref[:], w1_ref[:], preferred_element_type=jnp.float32)
        acc = acc + jnp.dot(msg, w2_ref[:], preferred_element_type=jnp.float32)
        o_ref[:] = jnp.maximum(acc + b_ref[:], 0.0)

    return pl.pallas_call(
        body,
        grid=(N // RB,),
        in_specs=[
            pl.BlockSpec((RB, D), lambda i: (i, 0)),
            pl.BlockSpec((RB, D), lambda i: (i, 0)),
            pl.BlockSpec((RB, L), lambda i: (i, 0)),
            pl.BlockSpec((D, Dout), lambda i: (0, 0)),
            pl.BlockSpec((D, Dout), lambda i: (0, 0)),
            pl.BlockSpec((1, Dout), lambda i: (0, 0)),
        ],
        out_specs=pl.BlockSpec((RB, Dout), lambda i: (i, 0)),
        out_shape=jax.ShapeDtypeStruct((N, Dout), jnp.float32),
    )(x, agg[:N], deg[:N], W1, W2, b2)


def kernel(ability, labels, W_wkr, b_wkr, W_tsk, b_tsk, edge_weight,
           src_tw, dst_tw, etype_tw, src_wt, dst_wt, etype_wt):
    n_wkr = ability.shape[0]
    n_tsk = labels.shape[0]

    agg_w, deg_w = _sc_gather_segsum(labels, src_tw, dst_tw, etype_tw,
                                     edge_weight, n_wkr)
    ability_new = _tc_update(ability, agg_w, deg_w, W_wkr, b_wkr)

    agg_t, deg_t = _sc_gather_segsum(ability_new, src_wt, dst_wt, etype_wt,
                                     edge_weight, n_tsk)
    labels_new = _tc_update(labels, agg_t, deg_t, W_tsk, b_tsk)

    return ability_new, labels_new


# speedup vs baseline: 6.0887x; 2.0360x over previous
import functools

import jax
import jax.numpy as jnp
from jax import lax
from jax.experimental import pallas as pl
from jax.experimental.pallas import tpu as pltpu
from jax.experimental.pallas import tpu_sc as plsc

NUM_CORES = 2
NUM_SUBCORES = 16
NSUB = NUM_CORES * NUM_SUBCORES
K = 128
L = 16


C = 256
G = C // L


def _sc_gather_segsum(table, src, dst, etype, ew, n_out):
    E = src.shape[0]
    D = table.shape[1]
    npw = ((n_out + NSUB - 1) // NSUB + 7) // 8 * 8
    n_pad = NSUB * npw

    e_pad = ((E + C - 1) // C) * C + 2 * C
    pad = e_pad - E
    src_p = jnp.pad(src, (0, pad))
    dst_p = jnp.pad(dst, (0, pad))
    et_p = jnp.pad(etype, (0, pad))

    bounds = (jnp.arange(NSUB + 1, dtype=jnp.int32) * npw).astype(jnp.int32)
    es = jnp.searchsorted(dst, bounds, side="left").astype(jnp.int32)
    es = jnp.pad(es, (0, 64 - (NSUB + 1)))

    R = ew.shape[0]
    r_pad = ((R + L - 1) // L) * L + L
    ew_p = jnp.pad(ew, (0, r_pad - R))

    mesh = plsc.VectorSubcoreMesh(
        core_axis_name="c", subcore_axis_name="s",
        num_cores=NUM_CORES, num_subcores=NUM_SUBCORES)

    @functools.partial(
        pl.kernel,
        mesh=mesh,
        out_type=(jax.ShapeDtypeStruct((n_pad, D), jnp.float32),
                  jax.ShapeDtypeStruct((n_pad, L), jnp.float32)),
        scratch_types=[
            pltpu.VMEM((C,), jnp.int32),
            pltpu.VMEM((C,), jnp.int32),
            pltpu.VMEM((C,), jnp.int32),
            pltpu.VMEM((C,), jnp.int32),
            pltpu.VMEM((C,), jnp.int32),
            pltpu.VMEM((C,), jnp.int32),
            pltpu.VMEM((C, D), jnp.float32),
            pltpu.VMEM((C, D), jnp.float32),
            pltpu.VMEM((npw, D), jnp.float32),
            pltpu.VMEM((npw, L), jnp.float32),
            pltpu.VMEM((64,), jnp.int32),
            pltpu.VMEM((r_pad,), jnp.float32),
            pltpu.SemaphoreType.DMA,
            pltpu.SemaphoreType.DMA,
            pltpu.SemaphoreType.DMA,
            pltpu.SemaphoreType.DMA,
            pltpu.SemaphoreType.DMA,
            pltpu.SemaphoreType.DMA,
        ],
    )
    def k(table_h, src_h, dst_h, et_h, ew_h, es_h, agg_h, deg_h,
          src0, src1, dst0, dst1, et0, et1, rows0, rows1,
          acc_v, deg_v, es_s, ew_s,
          sem_s0, sem_s1, sem_m0, sem_m1, sem_r0, sem_r1):
        srcb = (src0, src1)
        dstb = (dst0, dst1)
        etb = (et0, et1)
        rowsb = (rows0, rows1)
        sem_s = (sem_s0, sem_s1)
        sem_m = (sem_m0, sem_m1)
        sem_r = (sem_r0, sem_r1)

        wid = lax.axis_index("s") * NUM_CORES + lax.axis_index("c")
        nbase = wid * npw
        pltpu.sync_copy(es_h, es_s)
        pltpu.sync_copy(ew_h, ew_s)
        es_pair = es_s[pl.ds(wid, L)]
        estart = es_pair[0]
        eend = es_pair[1]

        zero = jnp.zeros((L,), jnp.float32)

        def zbody(i, carry):
            for j in range(D // L):
                acc_v[i, pl.ds(j * L, L)] = zero
            deg_v[i, pl.ds(0, L)] = zero
            return carry
        lax.fori_loop(0, npw, zbody, 0)

        e0a = (estart // 8) * 8
        nchunks = (eend - e0a + (C - 1)) // C

        def fire_src(chunk, b):
            cb = e0a + chunk * C
            pltpu.async_copy(src_h.at[pl.ds(cb, C)], srcb[b], sem_s[b])

        def drain_src(chunk, b):
            cb = e0a + chunk * C
            pltpu.make_async_copy(
                src_h.at[pl.ds(cb, C)], srcb[b], sem_s[b]).wait()

        def fire_meta(chunk, b):
            cb = e0a + chunk * C
            pltpu.async_copy(dst_h.at[pl.ds(cb, C)], dstb[b], sem_m[b])
            pltpu.async_copy(et_h.at[pl.ds(cb, C)], etb[b], sem_m[b])

        def drain_meta(chunk, b):
            cb = e0a + chunk * C
            pltpu.make_async_copy(
                dst_h.at[pl.ds(cb, C)], dstb[b], sem_m[b]).wait()
            pltpu.make_async_copy(
                et_h.at[pl.ds(cb, C)], etb[b], sem_m[b]).wait()

        def fire_rows(b):
            for j in range(C // K):
                pltpu.async_copy(
                    table_h.at[srcb[b].at[pl.ds(j * K, K)]],
                    rowsb[b].at[pl.ds(j * K, K)], sem_r[b])

        def drain_rows(b):
            for j in range(C // K):
                pltpu.make_async_copy(
                    table_h.at[srcb[b].at[pl.ds(j * K, K)]],
                    rowsb[b].at[pl.ds(j * K, K)], sem_r[b]).wait()

        def compute(c, b):
            cb = e0a + c * C
            lo_rel = estart - cb
            hi_rel = eend - cb
            rows_v = rowsb[b]
            dst_s = dstb[b]
            et_s = etb[b]
            ew_vec = ew_s[pl.ds(0, L)]

            def group_body(g, carry):
                base = g * L
                dst16 = dst_s[pl.ds(base, L)]
                et16 = et_s[pl.ds(base, L)]
                w16 = lax.gather(
                    ew_vec, et16[:, None],
                    lax.GatherDimensionNumbers(
                        offset_dims=(), collapsed_slice_dims=(0,),
                        start_index_map=(0,)),
                    slice_sizes=(1,),
                    mode=lax.GatherScatterMode.PROMISE_IN_BOUNDS)
                rel = lax.iota(jnp.int32, L) + base
                mask = (rel >= lo_rel) & (rel < hi_rel)
                wm = jnp.where(mask, w16, 0.0)
                ones = jnp.where(mask, 1.0, 0.0)
                dm = jnp.clip(dst16 - nbase, 0, npw - 1)
                d0 = dm[0]
                uniform = ((dst16[0] == dst16[L - 1])
                           & (base >= lo_rel) & (base + L <= hi_rel))

                def fast_path():
                    for j in range(D // L):
                        sl = pl.ds(j * L, L)
                        s = rows_v[base, sl] * jnp.full((L,), wm[0])
                        for kk in range(1, L):
                            s = s + rows_v[base + kk, sl] * jnp.full(
                                (L,), wm[kk])
                        acc_v[d0, sl] = acc_v[d0, sl] + s
                    deg_v[d0, pl.ds(0, L)] = (
                        deg_v[d0, pl.ds(0, L)] + float(L))

                def slow_path():
                    for kk in range(L):
                        d = dm[kk]
                        e = base + kk
                        wv = jnp.full((L,), wm[kk], jnp.float32)
                        deg_v[d, pl.ds(0, L)] = (
                            deg_v[d, pl.ds(0, L)]
                            + jnp.full((L,), ones[kk], jnp.float32))
                        for j in range(D // L):
                            sl = pl.ds(j * L, L)
                            acc_v[d, sl] = acc_v[d, sl] + rows_v[e, sl] * wv

                lax.cond(uniform, fast_path, slow_path)
                return carry

            g_lo = jnp.maximum(lo_rel, 0) // L
            g_hi = (jnp.minimum(hi_rel, C) + (L - 1)) // L
            lax.fori_loop(g_lo, g_hi, group_body, 0)

        @pl.when(nchunks >= 1)
        def _():
            fire_src(0, 0)

        @pl.when(nchunks >= 2)
        def _():
            fire_src(1, 1)

        @pl.when(nchunks >= 1)
        def _():
            drain_src(0, 0)
            fire_meta(0, 0)
            fire_rows(0)

        def pair_body(c2, carry):
            for b in range(2):
                c = c2 * 2 + b
                nb = 1 - b

                @pl.when(c < nchunks)
                def _():
                    @pl.when(c + 1 < nchunks)
                    def _():
                        fire_meta(c + 1, nb)
                        drain_src(c + 1, nb)
                        fire_rows(nb)

                    drain_rows(b)
                    drain_meta(c, b)

                    @pl.when(c + 2 < nchunks)
                    def _():
                        fire_src(c + 2, b)

                    compute(c, b)
            return carry
        lax.fori_loop(0, (nchunks + 1) // 2, pair_body, 0)

        pltpu.sync_copy(acc_v, agg_h.at[pl.ds(nbase, npw)])
        pltpu.sync_copy(deg_v, deg_h.at[pl.ds(nbase, npw)])

    return k(table, src_p, dst_p, et_p, ew_p, es)


def _tc_update(x, agg, deg, W, b):
    N, D = x.shape
    W1 = W[:D]
    W2 = W[D:]
    b2 = b.reshape(1, -1)
    Dout = W.shape[1]
    RB = 1000
    assert N % RB == 0

    def body(x_ref, a_ref, dg_ref, w1_ref, w2_ref, b_ref, o_ref):
        deg_col = jnp.maximum(dg_ref[:, 0:1], 1.0)
        msg = a_ref[:] / deg_col
        acc = jnp.dot(x_ref[:], w1_ref[:], preferred_element_type=jnp.float32)
        acc = acc + jnp.dot(msg, w2_ref[:], preferred_element_type=jnp.float32)
        o_ref[:] = jnp.maximum(acc + b_ref[:], 0.0)

    return pl.pallas_call(
        body,
        grid=(N // RB,),
        in_specs=[
            pl.BlockSpec((RB, D), lambda i: (i, 0)),
            pl.BlockSpec((RB, D), lambda i: (i, 0)),
            pl.BlockSpec((RB, L), lambda i: (i, 0)),
            pl.BlockSpec((D, Dout), lambda i: (0, 0)),
            pl.BlockSpec((D, Dout), lambda i: (0, 0)),
            pl.BlockSpec((1, Dout), lambda i: (0, 0)),
        ],
        out_specs=pl.BlockSpec((RB, Dout), lambda i: (i, 0)),
        out_shape=jax.ShapeDtypeStruct((N, Dout), jnp.float32),
    )(x, agg[:N], deg[:N], W1, W2, b2)


def kernel(ability, labels, W_wkr, b_wkr, W_tsk, b_tsk, edge_weight,
           src_tw, dst_tw, etype_tw, src_wt, dst_wt, etype_wt):
    n_wkr = ability.shape[0]
    n_tsk = labels.shape[0]

    agg_w, deg_w = _sc_gather_segsum(labels, src_tw, dst_tw, etype_tw,
                                     edge_weight, n_wkr)
    ability_new = _tc_update(ability, agg_w, deg_w, W_wkr, b_wkr)

    agg_t, deg_t = _sc_gather_segsum(ability_new, src_wt, dst_wt, etype_wt,
                                     edge_weight, n_tsk)
    labels_new = _tc_update(labels, agg_t, deg_t, W_tsk, b_tsk)

    return ability_new, labels_new


# speedup vs baseline: 6.5122x; 1.0696x over previous
import functools

import jax
import jax.numpy as jnp
from jax import lax
from jax.experimental import pallas as pl
from jax.experimental.pallas import tpu as pltpu
from jax.experimental.pallas import tpu_sc as plsc

NUM_CORES = 2
NUM_SUBCORES = 16
NSUB = NUM_CORES * NUM_SUBCORES
K = 128
L = 16


C = 256
G = C // L


def _sc_gather_segsum(table, src, dst, etype, ew, n_out):
    E = src.shape[0]
    D = table.shape[1]
    npw = ((n_out + NSUB - 1) // NSUB + 7) // 8 * 8
    n_pad = NSUB * npw

    e_pad = ((E + C - 1) // C) * C + 2 * C
    pad = e_pad - E
    src_p = jnp.pad(src, (0, pad))
    dst_p = jnp.pad(dst, (0, pad))
    et_p = jnp.pad(etype, (0, pad))

    bounds = (jnp.arange(NSUB + 1, dtype=jnp.int32) * npw).astype(jnp.int32)
    es = jnp.searchsorted(dst, bounds, side="left").astype(jnp.int32)
    es = jnp.pad(es, (0, 64 - (NSUB + 1)))

    R = ew.shape[0]
    r_pad = ((R + L - 1) // L) * L + L
    ew_p = jnp.pad(ew, (0, r_pad - R))

    mesh = plsc.VectorSubcoreMesh(
        core_axis_name="c", subcore_axis_name="s",
        num_cores=NUM_CORES, num_subcores=NUM_SUBCORES)

    @functools.partial(
        pl.kernel,
        mesh=mesh,
        out_type=(jax.ShapeDtypeStruct((n_pad, D), jnp.float32),
                  jax.ShapeDtypeStruct((n_pad, L), jnp.float32)),
        scratch_types=[
            pltpu.VMEM((C,), jnp.int32),
            pltpu.VMEM((C,), jnp.int32),
            pltpu.VMEM((C,), jnp.int32),
            pltpu.VMEM((C,), jnp.int32),
            pltpu.VMEM((C,), jnp.int32),
            pltpu.VMEM((C,), jnp.int32),
            pltpu.VMEM((C, D), jnp.float32),
            pltpu.VMEM((C, D), jnp.float32),
            pltpu.VMEM((npw, D), jnp.float32),
            pltpu.VMEM((npw, L), jnp.float32),
            pltpu.VMEM((64,), jnp.int32),
            pltpu.VMEM((r_pad,), jnp.float32),
            pltpu.SemaphoreType.DMA,
            pltpu.SemaphoreType.DMA,
            pltpu.SemaphoreType.DMA,
            pltpu.SemaphoreType.DMA,
            pltpu.SemaphoreType.DMA,
            pltpu.SemaphoreType.DMA,
        ],
    )
    def k(table_h, src_h, dst_h, et_h, ew_h, es_h, agg_h, deg_h,
          src0, src1, dst0, dst1, et0, et1, rows0, rows1,
          acc_v, deg_v, es_s, ew_s,
          sem_s0, sem_s1, sem_m0, sem_m1, sem_r0, sem_r1):
        srcb = (src0, src1)
        dstb = (dst0, dst1)
        etb = (et0, et1)
        rowsb = (rows0, rows1)
        sem_s = (sem_s0, sem_s1)
        sem_m = (sem_m0, sem_m1)
        sem_r = (sem_r0, sem_r1)

        wid = lax.axis_index("s") * NUM_CORES + lax.axis_index("c")
        nbase = wid * npw
        pltpu.sync_copy(es_h, es_s)
        pltpu.sync_copy(ew_h, ew_s)
        es_pair = es_s[pl.ds(wid, L)]
        estart = es_pair[0]
        eend = es_pair[1]

        zero = jnp.zeros((L,), jnp.float32)

        def zbody(i, carry):
            for j in range(D // L):
                acc_v[i, pl.ds(j * L, L)] = zero
            deg_v[i, pl.ds(0, L)] = zero
            return carry
        lax.fori_loop(0, npw, zbody, 0)

        e0a = (estart // 8) * 8
        nchunks = (eend - e0a + (C - 1)) // C

        def fire_src(chunk, b):
            cb = e0a + chunk * C
            pltpu.async_copy(src_h.at[pl.ds(cb, C)], srcb[b], sem_s[b])

        def drain_src(chunk, b):
            cb = e0a + chunk * C
            pltpu.make_async_copy(
                src_h.at[pl.ds(cb, C)], srcb[b], sem_s[b]).wait()

        def fire_meta(chunk, b):
            cb = e0a + chunk * C
            pltpu.async_copy(dst_h.at[pl.ds(cb, C)], dstb[b], sem_m[b])
            pltpu.async_copy(et_h.at[pl.ds(cb, C)], etb[b], sem_m[b])

        def drain_meta(chunk, b):
            cb = e0a + chunk * C
            pltpu.make_async_copy(
                dst_h.at[pl.ds(cb, C)], dstb[b], sem_m[b]).wait()
            pltpu.make_async_copy(
                et_h.at[pl.ds(cb, C)], etb[b], sem_m[b]).wait()

        def fire_rows(b):
            for j in range(C // K):
                pltpu.async_copy(
                    table_h.at[srcb[b].at[pl.ds(j * K, K)]],
                    rowsb[b].at[pl.ds(j * K, K)], sem_r[b])

        def drain_rows(b):
            for j in range(C // K):
                pltpu.make_async_copy(
                    table_h.at[srcb[b].at[pl.ds(j * K, K)]],
                    rowsb[b].at[pl.ds(j * K, K)], sem_r[b]).wait()

        def compute(c, b):
            cb = e0a + c * C
            lo_rel = estart - cb
            hi_rel = eend - cb
            rows_v = rowsb[b]
            dst_s = dstb[b]
            et_s = etb[b]
            ew_vec = ew_s[pl.ds(0, L)]

            def group_body(g, carry):
                base = g * L
                dst16 = dst_s[pl.ds(base, L)]
                et16 = et_s[pl.ds(base, L)]
                w16 = lax.gather(
                    ew_vec, et16[:, None],
                    lax.GatherDimensionNumbers(
                        offset_dims=(), collapsed_slice_dims=(0,),
                        start_index_map=(0,)),
                    slice_sizes=(1,),
                    mode=lax.GatherScatterMode.PROMISE_IN_BOUNDS)
                rel = lax.iota(jnp.int32, L) + base
                mask = (rel >= lo_rel) & (rel < hi_rel)
                wm = jnp.where(mask, w16, 0.0)
                ones = jnp.where(mask, 1.0, 0.0)
                dm = jnp.clip(dst16 - nbase, 0, npw - 1)
                d0 = dm[0]
                uniform = ((dst16[0] == dst16[L - 1])
                           & (base >= lo_rel) & (base + L <= hi_rel))

                wvecs = [jnp.take(wm, jnp.full((L,), kk, jnp.int32))
                         for kk in range(L)]

                def fast_path():
                    for j in range(D // L):
                        sl = pl.ds(j * L, L)
                        terms = [rows_v[base + kk, sl] * wvecs[kk]
                                 for kk in range(L)]
                        while len(terms) > 1:
                            terms = [terms[i] + terms[i + 1]
                                     for i in range(0, len(terms), 2)]
                        acc_v[d0, sl] = acc_v[d0, sl] + terms[0]
                    deg_v[d0, pl.ds(0, L)] = (
                        deg_v[d0, pl.ds(0, L)] + float(L))

                def slow_path():
                    for kk in range(L):
                        d = dm[kk]
                        e = base + kk
                        wv = wvecs[kk]
                        deg_v[d, pl.ds(0, L)] = (
                            deg_v[d, pl.ds(0, L)]
                            + jnp.full((L,), ones[kk], jnp.float32))
                        for j in range(D // L):
                            sl = pl.ds(j * L, L)
                            acc_v[d, sl] = acc_v[d, sl] + rows_v[e, sl] * wv

                lax.cond(uniform, fast_path, slow_path)
                return carry

            g_lo = jnp.maximum(lo_rel, 0) // L
            g_hi = (jnp.minimum(hi_rel, C) + (L - 1)) // L
            lax.fori_loop(g_lo, g_hi, group_body, 0)

        @pl.when(nchunks >= 1)
        def _():
            fire_src(0, 0)

        @pl.when(nchunks >= 2)
        def _():
            fire_src(1, 1)

        @pl.when(nchunks >= 1)
        def _():
            drain_src(0, 0)
            fire_meta(0, 0)
            fire_rows(0)

        def pair_body(c2, carry):
            for b in range(2):
                c = c2 * 2 + b
                nb = 1 - b

                @pl.when(c < nchunks)
                def _():
                    @pl.when(c + 1 < nchunks)
                    def _():
                        fire_meta(c + 1, nb)
                        drain_src(c + 1, nb)
                        fire_rows(nb)

                    drain_rows(b)
                    drain_meta(c, b)

                    @pl.when(c + 2 < nchunks)
                    def _():
                        fire_src(c + 2, b)

                    compute(c, b)
            return carry
        lax.fori_loop(0, (nchunks + 1) // 2, pair_body, 0)

        pltpu.sync_copy(acc_v, agg_h.at[pl.ds(nbase, npw)])
        pltpu.sync_copy(deg_v, deg_h.at[pl.ds(nbase, npw)])

    return k(table, src_p, dst_p, et_p, ew_p, es)


def _tc_update(x, agg, deg, W, b):
    N, D = x.shape
    W1 = W[:D]
    W2 = W[D:]
    b2 = b.reshape(1, -1)
    Dout = W.shape[1]
    RB = 1000
    assert N % RB == 0

    def body(x_ref, a_ref, dg_ref, w1_ref, w2_ref, b_ref, o_ref):
        deg_col = jnp.maximum(dg_ref[:, 0:1], 1.0)
        msg = a_ref[:] / deg_col
        acc = jnp.dot(x_ref[:], w1_ref[:], preferred_element_type=jnp.float32)
        acc = acc + jnp.dot(msg, w2_ref[:], preferred_element_type=jnp.float32)
        o_ref[:] = jnp.maximum(acc + b_ref[:], 0.0)

    return pl.pallas_call(
        body,
        grid=(N // RB,),
        in_specs=[
            pl.BlockSpec((RB, D), lambda i: (i, 0)),
            pl.BlockSpec((RB, D), lambda i: (i, 0)),
            pl.BlockSpec((RB, L), lambda i: (i, 0)),
            pl.BlockSpec((D, Dout), lambda i: (0, 0)),
            pl.BlockSpec((D, Dout), lambda i: (0, 0)),
            pl.BlockSpec((1, Dout), lambda i: (0, 0)),
        ],
        out_specs=pl.BlockSpec((RB, Dout), lambda i: (i, 0)),
        out_shape=jax.ShapeDtypeStruct((N, Dout), jnp.float32),
    )(x, agg[:N], deg[:N], W1, W2, b2)


def kernel(ability, labels, W_wkr, b_wkr, W_tsk, b_tsk, edge_weight,
           src_tw, dst_tw, etype_tw, src_wt, dst_wt, etype_wt):
    n_wkr = ability.shape[0]
    n_tsk = labels.shape[0]

    agg_w, deg_w = _sc_gather_segsum(labels, src_tw, dst_tw, etype_tw,
                                     edge_weight, n_wkr)
    ability_new = _tc_update(ability, agg_w, deg_w, W_wkr, b_wkr)

    agg_t, deg_t = _sc_gather_segsum(ability_new, src_wt, dst_wt, etype_wt,
                                     edge_weight, n_tsk)
    labels_new = _tc_update(labels, agg_t, deg_t, W_tsk, b_tsk)

    return ability_new, labels_new
